# Initial kernel scaffold; baseline (speedup 1.0000x reference)
import jax
import jax.numpy as jnp
from jax.experimental import pallas as pl


def kernel(ims, target_masks, long_range_inds, h0, W_backbone, b_backbone, W_k, b_k, W_q, b_q):
    raise NotImplementedError("write your pallas kernel here")


# speedup vs baseline: 37.9089x; 37.9089x over previous
import functools
import math

import jax
import jax.numpy as jnp
from jax import lax
from jax.experimental import pallas as pl
from jax.experimental.pallas import tpu as pltpu
from jax.experimental.pallas import tpu_sc as plsc

B, C, Wd, Hd = 4, 3, 64, 64
N = Wd * Hd
D, KQ, PD = 256, 132, 128
K = 26
QX = 256
NW = 32
RPW = (B * N) // NW
NCH = RPW // 128
Y_C = float(math.log((1.0 - 1e-4) / 1e-4))
SCALE = float(KQ) ** -0.5

_I = False


def _f1_body(imsp_ref, segc_ref, lr_ref, wf_ref, bb_ref, wk_ref, bk_ref,
             wq_ref, bq_ref, ksf_ref, qsfx_ref, lrg_ref):
    xp = imsp_ref[0]
    cols = jnp.concatenate(
        [xp[di:di + 64, dj:dj + 64, :]
         for di in range(3) for dj in range(3)], axis=-1)
    cols = cols.reshape(N, 27)
    feat = lax.dot_general(cols, wf_ref[...],
                           (((1,), (0,)), ((), ())))
    feat = feat + bb_ref[...]
    fg = feat.reshape(Wd, Hd, D)
    ss = jnp.sum(fg * fg, axis=1, keepdims=True)
    fg = fg / jnp.maximum(jnp.sqrt(ss), 1e-12)
    flat = fg.reshape(N, D)
    ksf = lax.dot_general(flat, wk_ref[...],
                          (((1,), (1,)), ((), ()))) + bk_ref[...]
    qsf = lax.dot_general(flat, wq_ref[...],
                          (((1,), (1,)), ((), ()))) + bq_ref[...]
    ksf_ref[0] = ksf
    qsfx_ref[0] = jnp.concatenate(
        [qsf, segc_ref[0], jnp.zeros((N, QX - KQ - 1), jnp.float32)], axis=1)
    b = pl.program_id(0)
    lrg_ref[0] = lr_ref[0] + b * N


def _front(imsp, segc, lr, wf, bb, wk, bk, wq, bq):
    full = lambda a: pl.BlockSpec(a.shape, lambda b: (0,) * a.ndim)
    return pl.pallas_call(
        _f1_body,
        grid=(B,),
        in_specs=[
            pl.BlockSpec((1, 66, 66, C), lambda b: (b, 0, 0, 0)),
            pl.BlockSpec((1, N, 1), lambda b: (b, 0, 0)),
            pl.BlockSpec((1, N, 1), lambda b: (b, 0, 0)),
            full(wf), full(bb), full(wk), full(bk), full(wq), full(bq),
        ],
        out_specs=[
            pl.BlockSpec((1, N, KQ), lambda b: (b, 0, 0)),
            pl.BlockSpec((1, N, QX), lambda b: (b, 0, 0)),
            pl.BlockSpec((1, N, 1), lambda b: (b, 0, 0)),
        ],
        out_shape=[
            jax.ShapeDtypeStruct((B, N, KQ), jnp.float32),
            jax.ShapeDtypeStruct((B, N, QX), jnp.float32),
            jax.ShapeDtypeStruct((B, N, 1), jnp.int32),
        ],
        interpret=_I,
    )(imsp, segc, lr, wf, bb, wk, bk, wq, bq)


def _h0_body(h0_ref, h_ref):
    x = h0_ref[0]
    m = jnp.max(x, axis=-1, keepdims=True)
    e = jnp.exp(x - m)
    h_ref[0] = e / jnp.sum(e, axis=-1, keepdims=True)


def _h0(h0):
    return pl.pallas_call(
        _h0_body,
        grid=(B,),
        in_specs=[pl.BlockSpec((1, N, PD), lambda b: (b, 0, 0))],
        out_specs=pl.BlockSpec((1, N, PD), lambda b: (b, 0, 0)),
        out_shape=jax.ShapeDtypeStruct((B, N, PD), jnp.float32),
        interpret=_I,
    )(h0)


def _reflect_pad2(g):
    top = jnp.concatenate([g[2:3], g[1:2]], axis=0)
    bot = jnp.concatenate([g[62:63], g[61:62]], axis=0)
    g = jnp.concatenate([top, g, bot], axis=0)
    left = jnp.concatenate([g[:, 2:3], g[:, 1:2]], axis=1)
    right = jnp.concatenate([g[:, 62:63], g[:, 61:62]], axis=1)
    return jnp.concatenate([left, g, right], axis=1)


def _f2_body(ksf_ref, qsfx_ref, qg_ref, segg_ref, hin_ref,
             lg_ref, adjt_ref, t25_ref, lossp_ref, lts_ref):
    ksg = ksf_ref[0].reshape(Wd, Hd, KQ)
    qsf = qsfx_ref[0][:, :KQ]
    qp = _reflect_pad2(qsf.reshape(Wd, Hd, KQ))
    sgg = segg_ref[0]
    sp = _reflect_pad2(sgg)
    qgg = qg_ref[0].reshape(Wd, Hd, QX)
    for tap in range(25):
        di, dj = tap // 5, tap % 5
        lts_ref[tap] = jnp.sum(ksg * qp[di:di + 64, dj:dj + 64, :],
                               axis=-1) * SCALE
    lts_ref[25] = jnp.sum(ksg * qgg[:, :, :KQ], axis=-1) * SCALE
    mg = lts_ref[0]
    for tap in range(1, K):
        mg = jnp.maximum(mg, lts_ref[tap])
    sglr = jnp.sum(qgg[:, :, KQ:KQ + 1], axis=-1)
    loss = jnp.zeros((), jnp.float32)
    for tap in range(K):
        if tap < 25:
            di, dj = tap // 5, tap % 5
            conn = sgg == sp[di:di + 64, dj:dj + 64]
        else:
            conn = sgg == sglr
        y = jnp.where(conn, Y_C, -Y_C)
        lt = lts_ref[tap]
        d = lt - y
        loss = loss + jnp.sum(d * d)
        lg_ref[tap, 0] = lt
        adjt_ref[tap, 0] = jnp.exp(lt - mg)
    lossp_ref[0] = loss.reshape(1, 1)
    hin3 = hin_ref[0].reshape(Wd, Hd, PD)
    a25 = jnp.exp(lts_ref[25] - mg)
    t25_ref[0] = (hin3 * a25[:, :, None]).reshape(N, PD)


def _f2(ksf, qsfx, qg, segg, hin):
    return pl.pallas_call(
        _f2_body,
        grid=(B,),
        in_specs=[
            pl.BlockSpec((1, N, KQ), lambda b: (b, 0, 0)),
            pl.BlockSpec((1, N, QX), lambda b: (b, 0, 0)),
            pl.BlockSpec((1, N, QX), lambda b: (b, 0, 0)),
            pl.BlockSpec((1, Wd, Hd), lambda b: (b, 0, 0)),
            pl.BlockSpec((1, N, PD), lambda b: (b, 0, 0)),
        ],
        out_specs=[
            pl.BlockSpec((K, 1, Wd, Hd), lambda b: (0, b, 0, 0)),
            pl.BlockSpec((K, 1, Wd, Hd), lambda b: (0, b, 0, 0)),
            pl.BlockSpec((1, N, PD), lambda b: (b, 0, 0)),
            pl.BlockSpec((1, 1, 1), lambda b: (b, 0, 0)),
        ],
        out_shape=[
            jax.ShapeDtypeStruct((K, B, Wd, Hd), jnp.float32),
            jax.ShapeDtypeStruct((K, B, Wd, Hd), jnp.float32),
            jax.ShapeDtypeStruct((B, N, PD), jnp.float32),
            jax.ShapeDtypeStruct((B, 1, 1), jnp.float32),
        ],
        scratch_shapes=[pltpu.VMEM((K, Wd, Hd), jnp.float32)],
        interpret=_I,
    )(ksf, qsfx, qg, segg, hin)


def _mesh():
    return plsc.VectorSubcoreMesh(core_axis_name="c", subcore_axis_name="s")


def _sc_gather(qsfx, lrg3):
    @functools.partial(
        pl.kernel,
        out_type=jax.ShapeDtypeStruct((B * N, QX), jnp.float32),
        mesh=_mesh(),
        scratch_types=[
            pltpu.VMEM((NCH, 128), jnp.int32),
            pltpu.VMEM((128, QX), jnp.float32),
            pltpu.SemaphoreType.DMA,
        ],
    )
    def k(qsfx_hbm, lrg_hbm, out_hbm, idx_v, rows_v, sem):
        w = lax.axis_index("c") * 16 + lax.axis_index("s")
        pltpu.sync_copy(lrg_hbm.at[w], idx_v)
        for j in range(NCH):
            pltpu.async_copy(qsfx_hbm.at[idx_v.at[j]], rows_v, sem).wait()
            pltpu.sync_copy(rows_v,
                            out_hbm.at[pl.ds(w * RPW + j * 128, 128)])

    return k(qsfx, lrg3)


def _sc_scatter(t25, lrl3, zrows):
    @functools.partial(
        pl.kernel,
        out_type=jax.ShapeDtypeStruct((B * N, PD), jnp.float32),
        mesh=_mesh(),
        scratch_types=[
            pltpu.VMEM((2, 128), jnp.int32),
            pltpu.VMEM((256, PD), jnp.float32),
            pltpu.VMEM_SHARED((N, PD), jnp.float32),
            pltpu.SemaphoreType.DMA,
        ],
    )
    def k(t25_hbm, lrl_hbm, z_hbm, out_hbm, idx_v, buf_v, acc_sh, sem):
        cid = lax.axis_index("c")
        sid = lax.axis_index("s")
        for p in range(2):
            b = 2 * cid + p
            pltpu.sync_copy(z_hbm.at[pl.ds(sid * 256, 256)],
                            acc_sh.at[pl.ds(sid * 256, 256)])
            pltpu.sync_copy(lrl_hbm.at[b * 16 + sid], idx_v)
            pltpu.sync_copy(t25_hbm.at[pl.ds(b * N + sid * 256, 256)], buf_v)
            plsc.subcore_barrier()
            for j in range(2):
                pltpu.sync_copy(buf_v.at[pl.ds(j * 128, 128)],
                                acc_sh.at[idx_v.at[j]], add=True)
            plsc.subcore_barrier()
            pltpu.sync_copy(acc_sh.at[pl.ds(sid * 256, 256)],
                            out_hbm.at[pl.ds(b * N + sid * 256, 256)])

    return k(t25, lrl3, zrows)


def _stencil_body(h_ref, adjt_ref, s_ref):
    h2g = h_ref[0].reshape(Wd, Hd, PD)
    accp = jnp.zeros((68, 68, PD), jnp.float32)
    for tap in range(25):
        di, dj = tap // 5, tap % 5
        wv = adjt_ref[tap, 0]
        t = h2g * wv[:, :, None]
        rparts = []
        if di:
            rparts.append(jnp.zeros((di, 64, PD), jnp.float32))
        rparts.append(t)
        if 4 - di:
            rparts.append(jnp.zeros((4 - di, 64, PD), jnp.float32))
        t = jnp.concatenate(rparts, axis=0) if len(rparts) > 1 else t
        cparts = []
        if dj:
            cparts.append(jnp.zeros((68, dj, PD), jnp.float32))
        cparts.append(t)
        if 4 - dj:
            cparts.append(jnp.zeros((68, 4 - dj, PD), jnp.float32))
        t = jnp.concatenate(cparts, axis=1) if len(cparts) > 1 else t
        accp = accp + t
    core = accp[2:66]
    core = jnp.concatenate([
        core[0:1],
        core[1:2] + accp[1:2],
        core[2:3] + accp[0:1],
        core[3:61],
        core[61:62] + accp[67:68],
        core[62:63] + accp[66:67],
        core[63:64],
    ], axis=0)
    out = core[:, 2:66]
    out = jnp.concatenate([
        out[:, 0:1],
        out[:, 1:2] + core[:, 1:2],
        out[:, 2:3] + core[:, 0:1],
        out[:, 3:61],
        out[:, 61:62] + core[:, 67:68],
        out[:, 62:63] + core[:, 66:67],
        out[:, 63:64],
    ], axis=1)
    s_ref[0] = out.reshape(N, PD)


def _stencil(h, adjt):
    return pl.pallas_call(
        _stencil_body,
        grid=(B,),
        in_specs=[
            pl.BlockSpec((1, N, PD), lambda b: (b, 0, 0)),
            pl.BlockSpec((K, 1, Wd, Hd), lambda b: (0, b, 0, 0)),
        ],
        out_specs=pl.BlockSpec((1, N, PD), lambda b: (b, 0, 0)),
        out_shape=jax.ShapeDtypeStruct((B, N, PD), jnp.float32),
        interpret=_I,
    )(h, adjt)


def _norm_body(s_ref, l_ref, w25_ref, h_ref, t25_ref):
    x = s_ref[0] + l_ref[0]
    nr = jnp.maximum(
        jnp.sqrt(jnp.sum(x * x, axis=-1, keepdims=True)), 1e-12)
    h = x / nr
    h_ref[0] = h
    wg = w25_ref[0, 0]
    t25_ref[0] = (h.reshape(Wd, Hd, PD) * wg[:, :, None]).reshape(N, PD)


def _normalize(s, l, w25):
    return pl.pallas_call(
        _norm_body,
        grid=(B,),
        in_specs=[
            pl.BlockSpec((1, N, PD), lambda b: (b, 0, 0)),
            pl.BlockSpec((1, N, PD), lambda b: (b, 0, 0)),
            pl.BlockSpec((1, 1, Wd, Hd), lambda b: (0, b, 0, 0)),
        ],
        out_specs=[
            pl.BlockSpec((1, N, PD), lambda b: (b, 0, 0)),
            pl.BlockSpec((1, N, PD), lambda b: (b, 0, 0)),
        ],
        out_shape=[
            jax.ShapeDtypeStruct((B, N, PD), jnp.float32),
            jax.ShapeDtypeStruct((B, N, PD), jnp.float32),
        ],
        interpret=_I,
    )(s, l, w25)


def kernel(ims, target_masks, long_range_inds, h0,
           W_backbone, b_backbone, W_k, b_k, W_q, b_q):
    wf = W_backbone.transpose(2, 3, 1, 0).reshape(27, D)
    imsp = jnp.pad(jnp.transpose(ims, (0, 2, 3, 1)),
                   ((0, 0), (1, 1), (1, 1), (0, 0)))
    segc = target_masks.astype(jnp.float32).reshape(B, N, 1)
    segg = target_masks.astype(jnp.float32).reshape(B, Wd, Hd)
    lr = long_range_inds.astype(jnp.int32)

    ksf, qsfx, lrg = _front(imsp, segc, lr, wf, b_backbone.reshape(1, D),
                            W_k, b_k.reshape(1, KQ), W_q, b_q.reshape(1, KQ))
    qg = _sc_gather(qsfx.reshape(B * N, QX), lrg.reshape(NW, NCH, 128))
    hin = _h0(h0)
    lgk, adjt, t25_0, lossp = _f2(ksf, qsfx, qg.reshape(B, N, QX), segg, hin)
    loss = jnp.sum(lossp) / float(B * N * K)
    logits = jnp.transpose(lgk.reshape(K, B, N), (1, 2, 0))

    lrl3 = lr.reshape(B * 16, 2, 128)
    zrows = jnp.zeros((N, PD), jnp.float32)
    w25 = adjt[25:26]

    def body(_, carry):
        h, t25 = carry
        s = _stencil(h, adjt)
        l = _sc_scatter(t25.reshape(B * N, PD), lrl3, zrows)
        return _normalize(s, l.reshape(B, N, PD), w25)

    h_fin, _ = lax.fori_loop(0, 72, body, (hin, t25_0))
    return loss, logits, h_fin


# speedup vs baseline: 40.4396x; 1.0668x over previous
import functools
import math

import jax
import jax.numpy as jnp
from jax import lax
from jax.experimental import pallas as pl
from jax.experimental.pallas import tpu as pltpu
from jax.experimental.pallas import tpu_sc as plsc

B, C, Wd, Hd = 4, 3, 64, 64
N = Wd * Hd
D, KQ, PD = 256, 132, 128
K = 26
QX = 256
NW = 32
RPW = (B * N) // NW
NCH = RPW // 128
Y_C = float(math.log((1.0 - 1e-4) / 1e-4))
SCALE = float(KQ) ** -0.5

_I = False


def _f1_body(imsp_ref, segc_ref, lr_ref, wf_ref, bb_ref, wk_ref, bk_ref,
             wq_ref, bq_ref, ksf_ref, qsfx_ref, lrg_ref):
    xp = imsp_ref[0]
    cols = jnp.concatenate(
        [xp[di:di + 64, dj:dj + 64, :]
         for di in range(3) for dj in range(3)], axis=-1)
    cols = cols.reshape(N, 27)
    feat = lax.dot_general(cols, wf_ref[...],
                           (((1,), (0,)), ((), ())))
    feat = feat + bb_ref[...]
    fg = feat.reshape(Wd, Hd, D)
    ss = jnp.sum(fg * fg, axis=1, keepdims=True)
    fg = fg / jnp.maximum(jnp.sqrt(ss), 1e-12)
    flat = fg.reshape(N, D)
    ksf = lax.dot_general(flat, wk_ref[...],
                          (((1,), (1,)), ((), ()))) + bk_ref[...]
    qsf = lax.dot_general(flat, wq_ref[...],
                          (((1,), (1,)), ((), ()))) + bq_ref[...]
    ksf_ref[0] = ksf
    qsfx_ref[0] = jnp.concatenate(
        [qsf, segc_ref[0], jnp.zeros((N, QX - KQ - 1), jnp.float32)], axis=1)
    b = pl.program_id(0)
    lrg_ref[0] = lr_ref[0] + b * N


def _front(imsp, segc, lr, wf, bb, wk, bk, wq, bq):
    full = lambda a: pl.BlockSpec(a.shape, lambda b: (0,) * a.ndim)
    return pl.pallas_call(
        _f1_body,
        grid=(B,),
        in_specs=[
            pl.BlockSpec((1, 66, 66, C), lambda b: (b, 0, 0, 0)),
            pl.BlockSpec((1, N, 1), lambda b: (b, 0, 0)),
            pl.BlockSpec((1, N, 1), lambda b: (b, 0, 0)),
            full(wf), full(bb), full(wk), full(bk), full(wq), full(bq),
        ],
        out_specs=[
            pl.BlockSpec((1, N, KQ), lambda b: (b, 0, 0)),
            pl.BlockSpec((1, N, QX), lambda b: (b, 0, 0)),
            pl.BlockSpec((1, N, 1), lambda b: (b, 0, 0)),
        ],
        out_shape=[
            jax.ShapeDtypeStruct((B, N, KQ), jnp.float32),
            jax.ShapeDtypeStruct((B, N, QX), jnp.float32),
            jax.ShapeDtypeStruct((B, N, 1), jnp.int32),
        ],
        interpret=_I,
    )(imsp, segc, lr, wf, bb, wk, bk, wq, bq)


def _h0_body(h0_ref, h_ref):
    x = h0_ref[0]
    m = jnp.max(x, axis=-1, keepdims=True)
    e = jnp.exp(x - m)
    h_ref[0] = e / jnp.sum(e, axis=-1, keepdims=True)


def _h0(h0):
    return pl.pallas_call(
        _h0_body,
        grid=(B,),
        in_specs=[pl.BlockSpec((1, N, PD), lambda b: (b, 0, 0))],
        out_specs=pl.BlockSpec((1, N, PD), lambda b: (b, 0, 0)),
        out_shape=jax.ShapeDtypeStruct((B, N, PD), jnp.float32),
        interpret=_I,
    )(h0)


def _reflect_pad2(g):
    top = jnp.concatenate([g[2:3], g[1:2]], axis=0)
    bot = jnp.concatenate([g[62:63], g[61:62]], axis=0)
    g = jnp.concatenate([top, g, bot], axis=0)
    left = jnp.concatenate([g[:, 2:3], g[:, 1:2]], axis=1)
    right = jnp.concatenate([g[:, 62:63], g[:, 61:62]], axis=1)
    return jnp.concatenate([left, g, right], axis=1)


def _f2_body(ksf_ref, qsfx_ref, qg_ref, segg_ref, hin_ref,
             lg_ref, adjt_ref, t25_ref, lossp_ref, lts_ref):
    ksg = ksf_ref[0].reshape(Wd, Hd, KQ)
    qsf = qsfx_ref[0][:, :KQ]
    qp = _reflect_pad2(qsf.reshape(Wd, Hd, KQ))
    sgg = segg_ref[0]
    sp = _reflect_pad2(sgg)
    qgg = qg_ref[0].reshape(Wd, Hd, QX)
    for tap in range(25):
        di, dj = tap // 5, tap % 5
        lts_ref[tap] = jnp.sum(ksg * qp[di:di + 64, dj:dj + 64, :],
                               axis=-1) * SCALE
    lts_ref[25] = jnp.sum(ksg * qgg[:, :, :KQ], axis=-1) * SCALE
    mg = lts_ref[0]
    for tap in range(1, K):
        mg = jnp.maximum(mg, lts_ref[tap])
    sglr = jnp.sum(qgg[:, :, KQ:KQ + 1], axis=-1)
    loss = jnp.zeros((), jnp.float32)
    for tap in range(K):
        if tap < 25:
            di, dj = tap // 5, tap % 5
            conn = sgg == sp[di:di + 64, dj:dj + 64]
        else:
            conn = sgg == sglr
        y = jnp.where(conn, Y_C, -Y_C)
        lt = lts_ref[tap]
        d = lt - y
        loss = loss + jnp.sum(d * d)
        lg_ref[tap, 0] = lt
        adjt_ref[tap, 0] = jnp.exp(lt - mg)
    lossp_ref[0] = loss.reshape(1, 1)
    hin3 = hin_ref[0].reshape(Wd, Hd, PD)
    a25 = jnp.exp(lts_ref[25] - mg)
    t25_ref[0] = (hin3 * a25[:, :, None]).reshape(N, PD)


def _f2(ksf, qsfx, qg, segg, hin):
    return pl.pallas_call(
        _f2_body,
        grid=(B,),
        in_specs=[
            pl.BlockSpec((1, N, KQ), lambda b: (b, 0, 0)),
            pl.BlockSpec((1, N, QX), lambda b: (b, 0, 0)),
            pl.BlockSpec((1, N, QX), lambda b: (b, 0, 0)),
            pl.BlockSpec((1, Wd, Hd), lambda b: (b, 0, 0)),
            pl.BlockSpec((1, N, PD), lambda b: (b, 0, 0)),
        ],
        out_specs=[
            pl.BlockSpec((K, 1, Wd, Hd), lambda b: (0, b, 0, 0)),
            pl.BlockSpec((K, 1, Wd, Hd), lambda b: (0, b, 0, 0)),
            pl.BlockSpec((1, N, PD), lambda b: (b, 0, 0)),
            pl.BlockSpec((1, 1, 1), lambda b: (b, 0, 0)),
        ],
        out_shape=[
            jax.ShapeDtypeStruct((K, B, Wd, Hd), jnp.float32),
            jax.ShapeDtypeStruct((K, B, Wd, Hd), jnp.float32),
            jax.ShapeDtypeStruct((B, N, PD), jnp.float32),
            jax.ShapeDtypeStruct((B, 1, 1), jnp.float32),
        ],
        scratch_shapes=[pltpu.VMEM((K, Wd, Hd), jnp.float32)],
        interpret=_I,
    )(ksf, qsfx, qg, segg, hin)


def _mesh():
    return plsc.VectorSubcoreMesh(core_axis_name="c", subcore_axis_name="s")


def _sc_gather(qsfx, lrg3):
    @functools.partial(
        pl.kernel,
        out_type=jax.ShapeDtypeStruct((B * N, QX), jnp.float32),
        mesh=_mesh(),
        scratch_types=[
            pltpu.VMEM((NCH, 128), jnp.int32),
            pltpu.VMEM((128, QX), jnp.float32),
            pltpu.SemaphoreType.DMA,
        ],
    )
    def k(qsfx_hbm, lrg_hbm, out_hbm, idx_v, rows_v, sem):
        w = lax.axis_index("c") * 16 + lax.axis_index("s")
        pltpu.sync_copy(lrg_hbm.at[w], idx_v)
        for j in range(NCH):
            pltpu.async_copy(qsfx_hbm.at[idx_v.at[j]], rows_v, sem).wait()
            pltpu.sync_copy(rows_v,
                            out_hbm.at[pl.ds(w * RPW + j * 128, 128)])

    return k(qsfx, lrg3)


def _sc_scatter(t25, lrl3, zrows):
    @functools.partial(
        pl.kernel,
        out_type=jax.ShapeDtypeStruct((B * N, PD), jnp.float32),
        mesh=_mesh(),
        scratch_types=[
            pltpu.VMEM((2, 128), jnp.int32),
            pltpu.VMEM((256, PD), jnp.float32),
            pltpu.VMEM_SHARED((N, PD), jnp.float32),
            pltpu.SemaphoreType.DMA,
        ],
    )
    def k(t25_hbm, lrl_hbm, z_hbm, out_hbm, idx_v, buf_v, acc_sh, sem):
        cid = lax.axis_index("c")
        sid = lax.axis_index("s")
        for p in range(2):
            b = 2 * cid + p
            pltpu.sync_copy(z_hbm.at[pl.ds(sid * 256, 256)],
                            acc_sh.at[pl.ds(sid * 256, 256)])
            pltpu.sync_copy(lrl_hbm.at[b * 16 + sid], idx_v)
            pltpu.sync_copy(t25_hbm.at[pl.ds(b * N + sid * 256, 256)], buf_v)
            plsc.subcore_barrier()
            for j in range(2):
                pltpu.sync_copy(buf_v.at[pl.ds(j * 128, 128)],
                                acc_sh.at[idx_v.at[j]], add=True)
            plsc.subcore_barrier()
            pltpu.sync_copy(acc_sh.at[pl.ds(sid * 256, 256)],
                            out_hbm.at[pl.ds(b * N + sid * 256, 256)])

    return k(t25, lrl3, zrows)


def _stencil_body(h_ref, adj_ref, s_ref, accp_ref):
    h2 = h_ref[0]
    accp_ref[...] = jnp.zeros((68, 68, PD), jnp.float32)
    for tap in range(25):
        di, dj = tap // 5, tap % 5
        wcol = adj_ref[0][:, tap:tap + 1]
        t = (h2 * wcol).reshape(Wd, Hd, PD)
        accp_ref[di:di + 64, dj:dj + 64, :] += t
    accp = accp_ref[...]
    core = accp[2:66]
    core = jnp.concatenate([
        core[0:1],
        core[1:2] + accp[1:2],
        core[2:3] + accp[0:1],
        core[3:61],
        core[61:62] + accp[67:68],
        core[62:63] + accp[66:67],
        core[63:64],
    ], axis=0)
    out = core[:, 2:66]
    out = jnp.concatenate([
        out[:, 0:1],
        out[:, 1:2] + core[:, 1:2],
        out[:, 2:3] + core[:, 0:1],
        out[:, 3:61],
        out[:, 61:62] + core[:, 67:68],
        out[:, 62:63] + core[:, 66:67],
        out[:, 63:64],
    ], axis=1)
    s_ref[0] = out.reshape(N, PD)


def _stencil(h, adj):
    return pl.pallas_call(
        _stencil_body,
        grid=(B,),
        in_specs=[
            pl.BlockSpec((1, N, PD), lambda b: (b, 0, 0)),
            pl.BlockSpec((1, N, K), lambda b: (b, 0, 0)),
        ],
        out_specs=pl.BlockSpec((1, N, PD), lambda b: (b, 0, 0)),
        out_shape=jax.ShapeDtypeStruct((B, N, PD), jnp.float32),
        scratch_shapes=[pltpu.VMEM((68, 68, PD), jnp.float32)],
        interpret=_I,
    )(h, adj)


def _norm_body(s_ref, l_ref, w25_ref, h_ref, t25_ref):
    x = s_ref[0] + l_ref[0]
    nr = jnp.maximum(
        jnp.sqrt(jnp.sum(x * x, axis=-1, keepdims=True)), 1e-12)
    h = x / nr
    h_ref[0] = h
    t25_ref[0] = h * w25_ref[0]


def _normalize(s, l, w25):
    return pl.pallas_call(
        _norm_body,
        grid=(B,),
        in_specs=[
            pl.BlockSpec((1, N, PD), lambda b: (b, 0, 0)),
            pl.BlockSpec((1, N, PD), lambda b: (b, 0, 0)),
            pl.BlockSpec((1, N, 1), lambda b: (b, 0, 0)),
        ],
        out_specs=[
            pl.BlockSpec((1, N, PD), lambda b: (b, 0, 0)),
            pl.BlockSpec((1, N, PD), lambda b: (b, 0, 0)),
        ],
        out_shape=[
            jax.ShapeDtypeStruct((B, N, PD), jnp.float32),
            jax.ShapeDtypeStruct((B, N, PD), jnp.float32),
        ],
        interpret=_I,
    )(s, l, w25)


def kernel(ims, target_masks, long_range_inds, h0,
           W_backbone, b_backbone, W_k, b_k, W_q, b_q):
    wf = W_backbone.transpose(2, 3, 1, 0).reshape(27, D)
    imsp = jnp.pad(jnp.transpose(ims, (0, 2, 3, 1)),
                   ((0, 0), (1, 1), (1, 1), (0, 0)))
    segc = target_masks.astype(jnp.float32).reshape(B, N, 1)
    segg = target_masks.astype(jnp.float32).reshape(B, Wd, Hd)
    lr = long_range_inds.astype(jnp.int32)

    ksf, qsfx, lrg = _front(imsp, segc, lr, wf, b_backbone.reshape(1, D),
                            W_k, b_k.reshape(1, KQ), W_q, b_q.reshape(1, KQ))
    qg = _sc_gather(qsfx.reshape(B * N, QX), lrg.reshape(NW, NCH, 128))
    hin = _h0(h0)
    lgk, adjtk, t25_0, lossp = _f2(ksf, qsfx, qg.reshape(B, N, QX), segg, hin)
    loss = jnp.sum(lossp) / float(B * N * K)
    logits = jnp.transpose(lgk.reshape(K, B, N), (1, 2, 0))
    adj = jnp.transpose(adjtk.reshape(K, B, N), (1, 2, 0))

    lrl3 = lr.reshape(B * 16, 2, 128)
    zrows = jnp.zeros((N, PD), jnp.float32)
    w25 = adj[:, :, 25:26]

    def body(_, carry):
        h, t25 = carry
        s = _stencil(h, adj)
        l = _sc_scatter(t25.reshape(B * N, PD), lrl3, zrows)
        return _normalize(s, l.reshape(B, N, PD), w25)

    h_fin, _ = lax.fori_loop(0, 72, body, (hin, t25_0))
    return loss, logits, h_fin
